# baseline (speedup 1.0000x reference)
import jax
import jax.numpy as jnp
from jax import lax
from jax.experimental import pallas as pl
from jax.experimental.pallas import tpu as pltpu
from jax.experimental.pallas import tpu_sc as plsc

BS, S, H = 16, 2048, 1024
N_PARA, N_SENT = 8, 64
N_OUT = 5
L = 16
HC = H // L
NQ = N_SENT // L


def _sc_kernel_body(table, sh, ph, pt, wmat, bvec, out,
                    idx_v, ph_v, pt_v, wb_v, wmat_v, b_v, rows_v, out_v, sem):
    nc = 2
    wid = lax.axis_index("s") * nc + lax.axis_index("c")

    @pl.when(wid < BS)
    def _():
        b = wid
        pltpu.sync_copy(sh.at[b], idx_v)
        pltpu.sync_copy(ph.at[b], ph_v)
        pltpu.sync_copy(pt.at[b], pt_v)
        pltpu.sync_copy(wmat, wmat_v)
        pltpu.sync_copy(bvec, b_v)

        lane = lax.iota(jnp.int32, L)
        zero_f = jnp.zeros((L,), jnp.float32)
        zero_i = jnp.zeros((L,), jnp.int32)

        one_f = jnp.ones((L,), jnp.float32)
        sq = [idx_v[pl.ds(q * L, L)] for q in range(NQ)]
        ph_row = ph_v[...]
        pt_row = pt_v[...]
        w_q = [zero_f for _ in range(NQ)]
        n_valid = zero_f
        for p in range(N_PARA):
            hp_s = jnp.sum(jnp.where(lane == p, ph_row, zero_i), axis=0)
            tp_s = jnp.sum(jnp.where(lane == p, pt_row, zero_i), axis=0)
            hp = jnp.full((L,), hp_s, jnp.int32)
            tp = jnp.full((L,), tp_s, jnp.int32)
            ok = (tp - hp) > 2
            cnt = zero_i
            vcnt = zero_i
            ms = []
            for q in range(NQ):
                m = ok & (hp <= sq[q]) & (sq[q] <= tp)
                ms.append(m)
                cnt = cnt + m.astype(jnp.int32)
                vcnt = vcnt + (m & (sq[q] != 0)).astype(jnp.int32)
            cnt_v = jnp.full((L,), jnp.sum(cnt, axis=0), jnp.int32).astype(jnp.float32)
            vcnt_v = jnp.full((L,), jnp.sum(vcnt, axis=0), jnp.int32)
            valid_v = jnp.where(vcnt_v > 0, one_f, zero_f)
            contrib = valid_v / jnp.maximum(cnt_v, one_f)
            n_valid = n_valid + valid_v
            for q in range(NQ):
                w_q[q] = w_q[q] + ms[q].astype(jnp.float32) * contrib
        hv_v = jnp.where(n_valid > 0, one_f, zero_f)
        svec = hv_v / jnp.maximum(n_valid, one_f)
        base = (one_f - hv_v) * (1.0 / N_SENT)
        for q in range(NQ):
            w_q[q] = w_q[q] * svec + base

        for s in range(N_SENT):
            ws = jnp.sum(jnp.where(lane == (s % L), w_q[s // L], zero_f), axis=0)
            wb_v[s, :] = jnp.full((L,), ws, jnp.float32)

        off = jnp.full((L,), b * S, jnp.int32)
        for q in range(NQ):
            idx_v[pl.ds(q * L, L)] = sq[q] + off
        pltpu.async_copy(table.at[idx_v], rows_v, sem).wait()

        def h_body(h, carry):
            def s_body(s, acc):
                return acc + wb_v[s, :] * rows_v[s, pl.ds(h * L, L)]
            acc = lax.fori_loop(0, N_SENT, s_body, zero_f)
            return tuple(c + acc * wmat_v[o, pl.ds(h * L, L)]
                         for o, c in enumerate(carry))

        outs = lax.fori_loop(0, HC, h_body, tuple(zero_f for _ in range(N_OUT)))

        res = b_v[...]
        for o in range(N_OUT):
            dvec = jnp.full((L,), jnp.sum(outs[o], axis=0), jnp.float32)
            res = res + jnp.where(lane == o, dvec, zero_f)
        out_v[...] = res
        pltpu.sync_copy(out_v, out.at[b])


@jax.jit
def kernel(backbone_outputs, attention_mask, paragraph_head_idxs, paragraph_tail_idxs,
           paragraph_attention_mask, sentence_head_idxs, sentence_tail_idxs,
           sentence_attention_mask, W, b):
    del attention_mask, paragraph_attention_mask, sentence_tail_idxs, sentence_attention_mask
    table = backbone_outputs.reshape(BS * S, H)
    sh = sentence_head_idxs.astype(jnp.int32)
    ph = jnp.zeros((BS, L), jnp.int32).at[:, :N_PARA].set(paragraph_head_idxs.astype(jnp.int32))
    pt = jnp.zeros((BS, L), jnp.int32).at[:, :N_PARA].set(paragraph_tail_idxs.astype(jnp.int32))
    b_pad = jnp.zeros((L,), jnp.float32).at[:N_OUT].set(b.astype(jnp.float32))

    mesh = plsc.VectorSubcoreMesh(core_axis_name="c", subcore_axis_name="s")
    out_pad = pl.kernel(
        _sc_kernel_body,
        mesh=mesh,
        compiler_params=pltpu.CompilerParams(needs_layout_passes=False),
        out_type=jax.ShapeDtypeStruct((BS, L), jnp.float32),
        scratch_types=[
            pltpu.VMEM((N_SENT,), jnp.int32),
            pltpu.VMEM((L,), jnp.int32),
            pltpu.VMEM((L,), jnp.int32),
            pltpu.VMEM((N_SENT, L), jnp.float32),
            pltpu.VMEM((N_OUT, H), jnp.float32),
            pltpu.VMEM((L,), jnp.float32),
            pltpu.VMEM((N_SENT, H), jnp.float32),
            pltpu.VMEM((L,), jnp.float32),
            pltpu.SemaphoreType.DMA,
        ],
    )(table, sh, ph, pt, W.astype(jnp.float32), b_pad)
    return out_pad[:, :N_OUT]


# speedup vs baseline: 1.0324x; 1.0324x over previous
import jax
import jax.numpy as jnp
from jax import lax
from jax.experimental import pallas as pl
from jax.experimental.pallas import tpu as pltpu
from jax.experimental.pallas import tpu_sc as plsc

BS, S, H = 16, 2048, 1024
N_PARA, N_SENT = 8, 64
N_OUT = 5
L = 16
HC = H // L
NQ = N_SENT // L


def _sc_kernel_body(table, sh, ph, pt, wmat, bvec, out,
                    idx_v, ph_v, pt_v, wb_v, wmat_v, b_v, rows_v, out_v, sem):
    wid = lax.axis_index("s")

    @pl.when(wid < BS)
    def _():
        b = wid
        pltpu.sync_copy(sh.at[b], idx_v)
        pltpu.sync_copy(ph.at[b], ph_v)
        pltpu.sync_copy(pt.at[b], pt_v)
        pltpu.sync_copy(wmat, wmat_v)
        pltpu.sync_copy(bvec, b_v)

        lane = lax.iota(jnp.int32, L)
        zero_f = jnp.zeros((L,), jnp.float32)
        zero_i = jnp.zeros((L,), jnp.int32)

        one_f = jnp.ones((L,), jnp.float32)
        sq = [idx_v[pl.ds(q * L, L)] for q in range(NQ)]
        ph_row = ph_v[...]
        pt_row = pt_v[...]
        w_q = [zero_f for _ in range(NQ)]
        n_valid = zero_f
        for p in range(N_PARA):
            hp_s = jnp.sum(jnp.where(lane == p, ph_row, zero_i), axis=0)
            tp_s = jnp.sum(jnp.where(lane == p, pt_row, zero_i), axis=0)
            hp = jnp.full((L,), hp_s, jnp.int32)
            tp = jnp.full((L,), tp_s, jnp.int32)
            ok = (tp - hp) > 2
            cnt = zero_i
            vcnt = zero_i
            ms = []
            for q in range(NQ):
                m = ok & (hp <= sq[q]) & (sq[q] <= tp)
                ms.append(m)
                cnt = cnt + m.astype(jnp.int32)
                vcnt = vcnt + (m & (sq[q] != 0)).astype(jnp.int32)
            cnt_v = jnp.full((L,), jnp.sum(cnt, axis=0), jnp.int32).astype(jnp.float32)
            vcnt_v = jnp.full((L,), jnp.sum(vcnt, axis=0), jnp.int32)
            valid_v = jnp.where(vcnt_v > 0, one_f, zero_f)
            contrib = valid_v / jnp.maximum(cnt_v, one_f)
            n_valid = n_valid + valid_v
            for q in range(NQ):
                w_q[q] = w_q[q] + ms[q].astype(jnp.float32) * contrib
        hv_v = jnp.where(n_valid > 0, one_f, zero_f)
        svec = hv_v / jnp.maximum(n_valid, one_f)
        base = (one_f - hv_v) * (1.0 / N_SENT)
        for q in range(NQ):
            w_q[q] = w_q[q] * svec + base

        for s in range(N_SENT):
            ws = jnp.sum(jnp.where(lane == (s % L), w_q[s // L], zero_f), axis=0)
            wb_v[s, :] = jnp.full((L,), ws, jnp.float32)

        off = jnp.full((L,), b * S, jnp.int32)
        for q in range(NQ):
            idx_v[pl.ds(q * L, L)] = sq[q] + off
        pltpu.async_copy(table.at[idx_v], rows_v, sem).wait()

        def h_body(h, carry):
            def s_body(s, acc):
                return acc + wb_v[s, :] * rows_v[s, pl.ds(h * L, L)]
            acc = lax.fori_loop(0, N_SENT, s_body, zero_f)
            return tuple(c + acc * wmat_v[o, pl.ds(h * L, L)]
                         for o, c in enumerate(carry))

        outs = lax.fori_loop(0, HC, h_body, tuple(zero_f for _ in range(N_OUT)))

        res = b_v[...]
        for o in range(N_OUT):
            dvec = jnp.full((L,), jnp.sum(outs[o], axis=0), jnp.float32)
            res = res + jnp.where(lane == o, dvec, zero_f)
        out_v[...] = res
        pltpu.sync_copy(out_v, out.at[b])


@jax.jit
def kernel(backbone_outputs, attention_mask, paragraph_head_idxs, paragraph_tail_idxs,
           paragraph_attention_mask, sentence_head_idxs, sentence_tail_idxs,
           sentence_attention_mask, W, b):
    del attention_mask, paragraph_attention_mask, sentence_tail_idxs, sentence_attention_mask
    table = backbone_outputs.reshape(BS * S, H)
    sh = sentence_head_idxs.astype(jnp.int32)
    ph = jnp.zeros((BS, L), jnp.int32).at[:, :N_PARA].set(paragraph_head_idxs.astype(jnp.int32))
    pt = jnp.zeros((BS, L), jnp.int32).at[:, :N_PARA].set(paragraph_tail_idxs.astype(jnp.int32))
    b_pad = jnp.zeros((L,), jnp.float32).at[:N_OUT].set(b.astype(jnp.float32))

    mesh = plsc.VectorSubcoreMesh(core_axis_name="c", subcore_axis_name="s", num_cores=1)
    out_pad = pl.kernel(
        _sc_kernel_body,
        mesh=mesh,
        compiler_params=pltpu.CompilerParams(needs_layout_passes=False),
        out_type=jax.ShapeDtypeStruct((BS, L), jnp.float32),
        scratch_types=[
            pltpu.VMEM((N_SENT,), jnp.int32),
            pltpu.VMEM((L,), jnp.int32),
            pltpu.VMEM((L,), jnp.int32),
            pltpu.VMEM((N_SENT, L), jnp.float32),
            pltpu.VMEM((N_OUT, H), jnp.float32),
            pltpu.VMEM((L,), jnp.float32),
            pltpu.VMEM((N_SENT, H), jnp.float32),
            pltpu.VMEM((L,), jnp.float32),
            pltpu.SemaphoreType.DMA,
        ],
    )(table, sh, ph, pt, W.astype(jnp.float32), b_pad)
    return out_pad[:, :N_OUT]


# speedup vs baseline: 1.6003x; 1.5500x over previous
import jax
import jax.numpy as jnp
from jax import lax
from jax.experimental import pallas as pl
from jax.experimental.pallas import tpu as pltpu
from jax.experimental.pallas import tpu_sc as plsc

BS, S, H = 16, 2048, 1024
N_PARA, N_SENT = 8, 64
N_OUT = 5
L = 16
HC = H // L
NQ = N_SENT // L
SPW = N_SENT // 2
CHUNK = 4


def _sc_kernel_body(table, packed, wmat, out,
                    pk_v, gidx_v, wb_v, wmat_v, rows_v, out_v, sem, wsem):
    c = lax.axis_index("c")
    s_id = lax.axis_index("s")
    b = s_id
    half = c

    wcp = pltpu.async_copy(wmat, wmat_v, wsem)
    pltpu.sync_copy(packed.at[b], pk_v)

    lane = lax.iota(jnp.int32, L)
    zero_f = jnp.zeros((L,), jnp.float32)
    zero_i = jnp.zeros((L,), jnp.int32)
    one_f = jnp.ones((L,), jnp.float32)

    off = jnp.full((L,), b * S, jnp.int32)
    hbase = half * SPW
    for j in range(SPW // L):
        gidx_v[pl.ds(j * L, L)] = pk_v[pl.ds(hbase + j * L, L)] + off
    gcp = pltpu.async_copy(table.at[gidx_v], rows_v, sem)

    sq = [pk_v[pl.ds(q * L, L)] for q in range(NQ)]
    hp_tp = pk_v[pl.ds(N_SENT, L)]
    w_q = [zero_f for _ in range(NQ)]
    n_valid = zero_f
    for p in range(N_PARA):
        hp_s = jnp.sum(jnp.where(lane == p, hp_tp, zero_i), axis=0)
        tp_s = jnp.sum(jnp.where(lane == (p + N_PARA), hp_tp, zero_i), axis=0)
        hp = jnp.full((L,), hp_s, jnp.int32)
        tp = jnp.full((L,), tp_s, jnp.int32)
        ok = (tp - hp) > 2
        cnt = zero_i
        vcnt = zero_i
        ms = []
        for q in range(NQ):
            m = ok & (hp <= sq[q]) & (sq[q] <= tp)
            ms.append(m)
            cnt = cnt + m.astype(jnp.int32)
            vcnt = vcnt + (m & (sq[q] != 0)).astype(jnp.int32)
        cnt_v = jnp.full((L,), jnp.sum(cnt, axis=0), jnp.int32).astype(jnp.float32)
        vcnt_v = jnp.full((L,), jnp.sum(vcnt, axis=0), jnp.int32)
        valid_v = jnp.where(vcnt_v > 0, one_f, zero_f)
        contrib = valid_v / jnp.maximum(cnt_v, one_f)
        n_valid = n_valid + valid_v
        for q in range(NQ):
            w_q[q] = w_q[q] + ms[q].astype(jnp.float32) * contrib
    hv_v = jnp.where(n_valid > 0, one_f, zero_f)
    svec = hv_v / jnp.maximum(n_valid, one_f)
    base = (one_f - hv_v) * (1.0 / N_SENT)
    for q in range(NQ):
        w_q[q] = w_q[q] * svec + base

    half0 = jnp.full((L,), half, jnp.int32) == 0
    for j in range(SPW):
        wq_sel = jnp.where(half0, w_q[j // L], w_q[NQ // 2 + j // L])
        ws = jnp.sum(jnp.where(lane == (j % L), wq_sel, zero_f), axis=0)
        wb_v[j, :] = jnp.full((L,), ws, jnp.float32)

    gcp.wait()
    wcp.wait()

    def hg_body(g, outs):
        def s_body(j, accs):
            wb = wb_v[j, :]
            return tuple(
                a + wb * rows_v[j, pl.ds((g * CHUNK + k) * L, L)]
                for k, a in enumerate(accs)
            )
        accs = lax.fori_loop(0, SPW, s_body, (zero_f,) * CHUNK)
        for k in range(CHUNK):
            outs = tuple(
                o + accs[k] * wmat_v[i, pl.ds((g * CHUNK + k) * L, L)]
                for i, o in enumerate(outs)
            )
        return outs

    outs = lax.fori_loop(0, HC // CHUNK, hg_body, (zero_f,) * N_OUT)

    res = zero_f
    for o in range(N_OUT):
        dvec = jnp.full((L,), jnp.sum(outs[o], axis=0), jnp.float32)
        res = res + jnp.where(lane == o, dvec, zero_f)
    out_v[...] = res
    pltpu.sync_copy(out_v, out.at[half * BS + b])


@jax.jit
def kernel(backbone_outputs, attention_mask, paragraph_head_idxs, paragraph_tail_idxs,
           paragraph_attention_mask, sentence_head_idxs, sentence_tail_idxs,
           sentence_attention_mask, W, b):
    del attention_mask, paragraph_attention_mask, sentence_tail_idxs, sentence_attention_mask
    table = backbone_outputs.reshape(BS * S, H)
    packed = jnp.concatenate(
        [sentence_head_idxs.astype(jnp.int32),
         paragraph_head_idxs.astype(jnp.int32),
         paragraph_tail_idxs.astype(jnp.int32)], axis=1)

    mesh = plsc.VectorSubcoreMesh(core_axis_name="c", subcore_axis_name="s")
    out_pad = pl.kernel(
        _sc_kernel_body,
        mesh=mesh,
        compiler_params=pltpu.CompilerParams(
            needs_layout_passes=False,
            disable_bounds_checks=True,
            disable_semaphore_checks=True,
        ),
        out_type=jax.ShapeDtypeStruct((2 * BS, L), jnp.float32),
        scratch_types=[
            pltpu.VMEM((N_SENT + L,), jnp.int32),
            pltpu.VMEM((SPW,), jnp.int32),
            pltpu.VMEM((SPW, L), jnp.float32),
            pltpu.VMEM((N_OUT, H), jnp.float32),
            pltpu.VMEM((SPW, H), jnp.float32),
            pltpu.VMEM((L,), jnp.float32),
            pltpu.SemaphoreType.DMA,
            pltpu.SemaphoreType.DMA,
        ],
    )(table, packed, W.astype(jnp.float32))
    halves = out_pad[:BS, :N_OUT] + out_pad[BS:, :N_OUT]
    return halves + b.astype(jnp.float32)[None, :]


# speedup vs baseline: 1.6176x; 1.0108x over previous
import jax
import jax.numpy as jnp
from jax import lax
from jax.experimental import pallas as pl
from jax.experimental.pallas import tpu as pltpu
from jax.experimental.pallas import tpu_sc as plsc

BS, S, H = 16, 2048, 1024
N_PARA, N_SENT = 8, 64
N_OUT = 5
L = 16
HC = H // L
NQ = N_SENT // L
SPW = N_SENT // 2
CHUNK = 4


def _sc_kernel_body(table, packed, wmat, out,
                    pk_v, gidx_v, wb_v, wmat_v, rows_v, out_v, sem, wsem):
    c = lax.axis_index("c")
    s_id = lax.axis_index("s")
    b = s_id
    half = c

    wcp = pltpu.async_copy(wmat, wmat_v, wsem)
    pltpu.sync_copy(packed.at[b], pk_v)

    lane = lax.iota(jnp.int32, L)
    zero_f = jnp.zeros((L,), jnp.float32)
    zero_i = jnp.zeros((L,), jnp.int32)
    one_f = jnp.ones((L,), jnp.float32)

    off = jnp.full((L,), b * S, jnp.int32)
    hbase = half * SPW
    for j in range(SPW // L):
        gidx_v[pl.ds(j * L, L)] = pk_v[pl.ds(hbase + j * L, L)] + off
    gcp = pltpu.async_copy(table.at[gidx_v], rows_v, sem)

    sq = [pk_v[pl.ds(q * L, L)] for q in range(NQ)]
    hp_tp = pk_v[pl.ds(N_SENT, L)]
    w_q = [zero_f for _ in range(NQ)]
    n_valid = zero_f
    for p in range(N_PARA):
        hp_s = jnp.sum(jnp.where(lane == p, hp_tp, zero_i), axis=0)
        tp_s = jnp.sum(jnp.where(lane == (p + N_PARA), hp_tp, zero_i), axis=0)
        hp = jnp.full((L,), hp_s, jnp.int32)
        tp = jnp.full((L,), tp_s, jnp.int32)
        ok = (tp - hp) > 2
        cnt = zero_i
        vcnt = zero_i
        ms = []
        for q in range(NQ):
            m = ok & (hp <= sq[q]) & (sq[q] <= tp)
            ms.append(m)
            cnt = cnt + m.astype(jnp.int32)
            vcnt = vcnt + (m & (sq[q] != 0)).astype(jnp.int32)
        cnt_v = jnp.full((L,), jnp.sum(cnt, axis=0), jnp.int32).astype(jnp.float32)
        vcnt_v = jnp.full((L,), jnp.sum(vcnt, axis=0), jnp.int32)
        valid_v = jnp.where(vcnt_v > 0, one_f, zero_f)
        contrib = valid_v / jnp.maximum(cnt_v, one_f)
        n_valid = n_valid + valid_v
        for q in range(NQ):
            w_q[q] = w_q[q] + ms[q].astype(jnp.float32) * contrib
    hv_v = jnp.where(n_valid > 0, one_f, zero_f)
    svec = hv_v / jnp.maximum(n_valid, one_f)
    base = (one_f - hv_v) * (1.0 / N_SENT)
    for q in range(NQ):
        w_q[q] = w_q[q] * svec + base

    half0 = jnp.full((L,), half, jnp.int32) == 0
    for j in range(SPW):
        wq_sel = jnp.where(half0, w_q[j // L], w_q[NQ // 2 + j // L])
        ws = jnp.sum(jnp.where(lane == (j % L), wq_sel, zero_f), axis=0)
        wb_v[j, :] = jnp.full((L,), ws, jnp.float32)

    gcp.wait()
    wcp.wait()

    def hg_body(g, outs):
        def s_body(j, accs):
            wb = wb_v[j, :]
            return tuple(
                a + wb * rows_v[j, pl.ds((g * CHUNK + k) * L, L)]
                for k, a in enumerate(accs)
            )
        accs = lax.fori_loop(0, SPW, s_body, (zero_f,) * CHUNK)
        for k in range(CHUNK):
            outs = tuple(
                o + accs[k] * wmat_v[i, pl.ds((g * CHUNK + k) * L, L)]
                for i, o in enumerate(outs)
            )
        return outs

    outs = lax.fori_loop(0, HC // CHUNK, hg_body, (zero_f,) * N_OUT)

    res = zero_f
    for o in range(N_OUT):
        dvec = jnp.full((L,), jnp.sum(outs[o], axis=0), jnp.float32)
        res = res + jnp.where(lane == o, dvec, zero_f)
    out_v[...] = res
    pltpu.sync_copy(out_v, out.at[half * BS + b])


@jax.jit
def kernel(backbone_outputs, attention_mask, paragraph_head_idxs, paragraph_tail_idxs,
           paragraph_attention_mask, sentence_head_idxs, sentence_tail_idxs,
           sentence_attention_mask, W, b):
    del attention_mask, paragraph_attention_mask, sentence_tail_idxs, sentence_attention_mask
    table = backbone_outputs.reshape(BS * S, H)
    packed = jnp.concatenate(
        [sentence_head_idxs.astype(jnp.int32),
         paragraph_head_idxs.astype(jnp.int32),
         paragraph_tail_idxs.astype(jnp.int32)], axis=1)

    mesh = plsc.VectorSubcoreMesh(core_axis_name="c", subcore_axis_name="s")
    out_pad = pl.kernel(
        _sc_kernel_body,
        mesh=mesh,
        compiler_params=pltpu.CompilerParams(
            needs_layout_passes=False,
            disable_bounds_checks=True,
            disable_semaphore_checks=True,
            skip_device_barrier=True,
        ),
        out_type=jax.ShapeDtypeStruct((2 * BS, L), jnp.float32),
        scratch_types=[
            pltpu.VMEM((N_SENT + L,), jnp.int32),
            pltpu.VMEM((SPW,), jnp.int32),
            pltpu.VMEM((SPW, L), jnp.float32),
            pltpu.VMEM((N_OUT, H), jnp.float32),
            pltpu.VMEM((SPW, H), jnp.float32),
            pltpu.VMEM((L,), jnp.float32),
            pltpu.SemaphoreType.DMA,
            pltpu.SemaphoreType.DMA,
        ],
    )(table, packed, W.astype(jnp.float32))
    halves = out_pad[:BS, :N_OUT] + out_pad[BS:, :N_OUT]
    return halves + b.astype(jnp.float32)[None, :]
